# Initial kernel scaffold; baseline (speedup 1.0000x reference)
import jax
import jax.numpy as jnp
from jax.experimental import pallas as pl


def kernel(features, edge_w, W_bond, b_bond, W1, b1, W2, b2, bn1_gamma, bn1_beta, bn2_gamma, bn2_beta, edge_index):
    raise NotImplementedError("write your pallas kernel here")


# speedup vs baseline: 4.8479x; 4.8479x over previous
import jax
import jax.numpy as jnp
from jax import lax
from jax.experimental import pallas as pl
from jax.experimental.pallas import tpu as pltpu
from jax.experimental.pallas import tpu_sc as plsc

N = 10000
E = 320000
H = 128
DE = 16
DEA = 32

NC = 2
NS = 16
NW = NC * NS
EW = E // NW
B = 100
RPC = 2
CHUNK = B * RPC
NCHUNK = EW // CHUNK
ROWS_PER_W = EW // B
STRIPE = 624
TAIL = N - NS * STRIPE

_MESH = plsc.VectorSubcoreMesh(core_axis_name="c", subcore_axis_name="s")


def _h1_body(feat_hbm, src_hbm, dst_hbm,
             h1_out,
             src_v, dst_v, rows_v, zrow_v,
             h1_acc, sem):
    c = lax.axis_index("c")
    s = lax.axis_index("s")
    wid = s * NC + c

    zeros16 = jnp.zeros((16,), jnp.float32)

    for r in range(16):
        for q in range(H // 16):
            zrow_v[r, q * 16:(q + 1) * 16] = zeros16

    r0 = s * STRIPE

    def _zero_acc(i, carry):
        off = pl.multiple_of(r0 + i * 16, 8)
        pltpu.sync_copy(zrow_v, h1_acc.at[pl.ds(off, 16)])
        return carry
    lax.fori_loop(0, STRIPE // 16, _zero_acc, None)

    @pl.when(s == NS - 1)
    def _zero_tail():
        pltpu.sync_copy(zrow_v, h1_acc.at[pl.ds(NS * STRIPE, TAIL)])

    plsc.subcore_barrier()

    wrow0 = wid * ROWS_PER_W

    def _chunk(g, carry):
        row0 = wrow0 + g * RPC
        pltpu.sync_copy(src_hbm.at[pl.ds(row0, RPC)], src_v)
        pltpu.sync_copy(dst_hbm.at[pl.ds(row0, RPC)], dst_v)
        cps = [pltpu.async_copy(feat_hbm.at[src_v.at[j]],
                                rows_v.at[pl.ds(j * B, B)], sem)
               for j in range(RPC)]
        for cp in cps:
            cp.wait()
        for j in range(RPC):
            pltpu.sync_copy(rows_v.at[pl.ds(j * B, B)],
                            h1_acc.at[dst_v.at[j]], add=True)
        return carry
    lax.fori_loop(0, NCHUNK, _chunk, None)

    plsc.subcore_barrier()

    out0 = pl.multiple_of(c * N + r0, 8)
    pltpu.sync_copy(h1_acc.at[pl.ds(r0, STRIPE)],
                    h1_out.at[pl.ds(out0, STRIPE)])

    @pl.when(s == NS - 1)
    def _copy_tail():
        t0 = pl.multiple_of(c * N + NS * STRIPE, 8)
        pltpu.sync_copy(h1_acc.at[pl.ds(NS * STRIPE, TAIL)],
                        h1_out.at[pl.ds(t0, TAIL)])


def _sew_body(dst_hbm, ew_hbm,
              sew_out,
              dst_v, ew_v, zsew_v,
              sew_acc, sem):
    c = lax.axis_index("c")
    s = lax.axis_index("s")
    wid = s * NC + c

    zeros16 = jnp.zeros((16,), jnp.float32)
    for r in range(16):
        for q in range(DEA // 16):
            zsew_v[r, q * 16:(q + 1) * 16] = zeros16

    r0 = s * STRIPE

    def _zero_acc(i, carry):
        off = pl.multiple_of(r0 + i * 16, 8)
        pltpu.sync_copy(zsew_v, sew_acc.at[pl.ds(off, 16)])
        return carry
    lax.fori_loop(0, STRIPE // 16, _zero_acc, None)

    @pl.when(s == NS - 1)
    def _zero_tail():
        pltpu.sync_copy(zsew_v, sew_acc.at[pl.ds(NS * STRIPE, TAIL)])

    plsc.subcore_barrier()

    wrow0 = wid * ROWS_PER_W
    webase = wid * EW

    def _chunk(g, carry):
        row0 = wrow0 + g * RPC
        e0 = pl.multiple_of(webase + g * CHUNK, 8)
        pltpu.sync_copy(dst_hbm.at[pl.ds(row0, RPC)], dst_v)
        pltpu.sync_copy(ew_hbm.at[pl.ds(e0, CHUNK)], ew_v)
        for j in range(RPC):
            pltpu.sync_copy(ew_v.at[pl.ds(j * B, B)],
                            sew_acc.at[dst_v.at[j]], add=True)
        return carry
    lax.fori_loop(0, NCHUNK, _chunk, None)

    plsc.subcore_barrier()

    out0 = pl.multiple_of(c * N + r0, 8)
    pltpu.sync_copy(sew_acc.at[pl.ds(r0, STRIPE)],
                    sew_out.at[pl.ds(out0, STRIPE)])

    @pl.when(s == NS - 1)
    def _copy_tail():
        t0 = pl.multiple_of(c * N + NS * STRIPE, 8)
        pltpu.sync_copy(sew_acc.at[pl.ds(NS * STRIPE, TAIL)],
                        sew_out.at[pl.ds(t0, TAIL)])


def _dense_body(h1p_ref, sewp_ref, feat_ref,
                waug_ref, w1_ref, b1_ref, w2_ref, b2_ref,
                g1_ref, be1_ref, g2_ref, be2_ref, out_ref):
    f32 = jnp.float32
    h1 = h1p_ref[0] + h1p_ref[1]
    sew = sewp_ref[0] + sewp_ref[1]

    dn_t = (((1,), (1,)), ((), ()))
    dn_n = (((1,), (0,)), ((), ()))
    h2 = lax.dot_general(sew, waug_ref[...], dn_n,
                         preferred_element_type=f32,
                         precision=lax.Precision.HIGHEST)
    x = lax.dot_general(h1, w1_ref[...], dn_t,
                        preferred_element_type=f32,
                        precision=lax.Precision.HIGHEST)
    x = x + b1_ref[...] + h2

    m1 = jnp.mean(x, axis=0)
    v1 = jnp.mean((x - m1) ** 2, axis=0)
    x = (x - m1) * lax.rsqrt(v1 + 1e-5) * g1_ref[...] + be1_ref[...]
    x = jnp.maximum(x, 0.0)

    y = lax.dot_general(x, w2_ref[...], dn_t,
                        preferred_element_type=f32,
                        precision=lax.Precision.HIGHEST)
    y = y + b2_ref[...] + feat_ref[...]

    m2 = jnp.mean(y, axis=0)
    v2 = jnp.mean((y - m2) ** 2, axis=0)
    y = (y - m2) * lax.rsqrt(v2 + 1e-5) * g2_ref[...] + be2_ref[...]
    out_ref[...] = jnp.maximum(y, 0.0)


@jax.jit
def kernel(features, edge_w, W_bond, b_bond, W1, b1, W2, b2,
           bn1_gamma, bn1_beta, bn2_gamma, bn2_beta, edge_index):
    src2 = edge_index[0].astype(jnp.int32).reshape(E // B, B)
    dst2 = edge_index[1].astype(jnp.int32).reshape(E // B, B)
    ew_aug = jnp.concatenate(
        [edge_w, jnp.ones((E, 1), jnp.float32),
         jnp.zeros((E, DEA - DE - 1), jnp.float32)], axis=1)
    waug = jnp.concatenate(
        [W_bond.T, b_bond[None, :], jnp.zeros((DEA - DE - 1, H), jnp.float32)],
        axis=0)

    h1_fn = pl.kernel(
        _h1_body,
        out_type=jax.ShapeDtypeStruct((NC * N, H), jnp.float32),
        mesh=_MESH,
        compiler_params=pltpu.CompilerParams(use_tc_tiling_on_sc=False),
        scratch_types=[
            pltpu.VMEM((RPC, B), jnp.int32),
            pltpu.VMEM((RPC, B), jnp.int32),
            pltpu.VMEM((CHUNK, H), jnp.float32),
            pltpu.VMEM((16, H), jnp.float32),
            pltpu.VMEM_SHARED((N, H), jnp.float32),
            pltpu.SemaphoreType.DMA,
        ],
    )
    h1p = h1_fn(features, src2, dst2)

    sew_fn = pl.kernel(
        _sew_body,
        out_type=jax.ShapeDtypeStruct((NC * N, DEA), jnp.float32),
        mesh=_MESH,
        compiler_params=pltpu.CompilerParams(use_tc_tiling_on_sc=False),
        scratch_types=[
            pltpu.VMEM((RPC, B), jnp.int32),
            pltpu.VMEM((CHUNK, DEA), jnp.float32),
            pltpu.VMEM((16, DEA), jnp.float32),
            pltpu.VMEM_SHARED((N, DEA), jnp.float32),
            pltpu.SemaphoreType.DMA,
        ],
    )
    sewp = sew_fn(dst2, ew_aug)

    out = pl.pallas_call(
        _dense_body,
        out_shape=jax.ShapeDtypeStruct((N, H), jnp.float32),
    )(h1p.reshape(NC, N, H), sewp.reshape(NC, N, DEA),
      features, waug, W1, b1, W2, b2,
      bn1_gamma, bn1_beta, bn2_gamma, bn2_beta)
    return out


# speedup vs baseline: 6.4448x; 1.3294x over previous
import jax
import jax.numpy as jnp
from jax import lax
from jax.experimental import pallas as pl
from jax.experimental.pallas import tpu as pltpu
from jax.experimental.pallas import tpu_sc as plsc

N = 10000
E = 320000
H = 128
DE = 16

NC = 2
NS = 16
NW = NC * NS
EW = E // NW
B = 125
NCHUNK = EW // B
ROWS_PER_W = EW // B
STRIPE = 624
TAIL = N - NS * STRIPE

_MESH = plsc.VectorSubcoreMesh(core_axis_name="c", subcore_axis_name="s")


def _h1_body(feat_hbm, src_hbm, dst_hbm,
             h1_out,
             src_v, dst_v, rows_v, zrow_v,
             h1_acc, sem):
    c = lax.axis_index("c")
    s = lax.axis_index("s")
    wid = s * NC + c

    zeros16 = jnp.zeros((16,), jnp.float32)

    for r in range(16):
        for q in range(H // 16):
            zrow_v[r, q * 16:(q + 1) * 16] = zeros16

    r0 = s * STRIPE

    def _zero_acc(i, carry):
        off = pl.multiple_of(r0 + i * 16, 8)
        pltpu.sync_copy(zrow_v, h1_acc.at[pl.ds(off, 16)])
        return carry
    lax.fori_loop(0, STRIPE // 16, _zero_acc, None)

    @pl.when(s == NS - 1)
    def _zero_tail():
        pltpu.sync_copy(zrow_v, h1_acc.at[pl.ds(NS * STRIPE, TAIL)])

    plsc.subcore_barrier()

    wrow0 = wid * ROWS_PER_W

    def _fire(g):
        bo = lax.rem(g, 2)
        row0 = wrow0 + g
        pltpu.sync_copy(src_hbm.at[pl.ds(row0, 1)], src_v.at[pl.ds(bo, 1)])
        pltpu.sync_copy(dst_hbm.at[pl.ds(row0, 1)], dst_v.at[pl.ds(bo, 1)])
        pltpu.async_copy(feat_hbm.at[src_v.at[bo]],
                         rows_v.at[pl.ds(bo * B, B)], sem)

    _fire(0)

    def _chunk(g, carry):
        @pl.when(g + 1 < NCHUNK)
        def _prefetch():
            _fire(g + 1)
        bo = lax.rem(g, 2)
        pltpu.make_async_copy(feat_hbm.at[src_v.at[bo]],
                              rows_v.at[pl.ds(bo * B, B)], sem).wait()
        pltpu.sync_copy(rows_v.at[pl.ds(bo * B, B)],
                        h1_acc.at[dst_v.at[bo]], add=True)
        return carry
    lax.fori_loop(0, NCHUNK, _chunk, None)

    plsc.subcore_barrier()

    out0 = pl.multiple_of(c * N + r0, 8)
    pltpu.sync_copy(h1_acc.at[pl.ds(r0, STRIPE)],
                    h1_out.at[pl.ds(out0, STRIPE)])

    @pl.when(s == NS - 1)
    def _copy_tail():
        t0 = pl.multiple_of(c * N + NS * STRIPE, 8)
        pltpu.sync_copy(h1_acc.at[pl.ds(NS * STRIPE, TAIL)],
                        h1_out.at[pl.ds(t0, TAIL)])


def _sew_body(dst_hbm, ew_hbm,
              sew_out, deg_out,
              dst_v, ew_v, ones_v, zsew_v,
              sew_acc, deg_acc, sem):
    c = lax.axis_index("c")
    s = lax.axis_index("s")
    wid = s * NC + c

    zeros16 = jnp.zeros((16,), jnp.float32)
    ones16 = jnp.full((16,), 1.0, jnp.float32)
    for r in range(16):
        zsew_v[r, 0:16] = zeros16
    for r in range(B):
        ones_v[r, 0:16] = ones16

    r0 = s * STRIPE

    def _zero_acc(i, carry):
        off = pl.multiple_of(r0 + i * 16, 8)
        pltpu.sync_copy(zsew_v, sew_acc.at[pl.ds(off, 16)])
        pltpu.sync_copy(zsew_v, deg_acc.at[pl.ds(off, 16)])
        return carry
    lax.fori_loop(0, STRIPE // 16, _zero_acc, None)

    @pl.when(s == NS - 1)
    def _zero_tail():
        pltpu.sync_copy(zsew_v, sew_acc.at[pl.ds(NS * STRIPE, TAIL)])
        pltpu.sync_copy(zsew_v, deg_acc.at[pl.ds(NS * STRIPE, TAIL)])

    plsc.subcore_barrier()

    wrow0 = wid * ROWS_PER_W
    webase = wid * EW

    def _fire(g):
        bo = lax.rem(g, 2)
        row0 = wrow0 + g
        e0 = webase + g * B
        pltpu.sync_copy(dst_hbm.at[pl.ds(row0, 1)], dst_v.at[pl.ds(bo, 1)])
        pltpu.async_copy(ew_hbm.at[pl.ds(e0, B)],
                         ew_v.at[pl.ds(bo * B, B)], sem)

    _fire(0)

    def _chunk(g, carry):
        @pl.when(g + 1 < NCHUNK)
        def _prefetch():
            _fire(g + 1)
        bo = lax.rem(g, 2)
        e0 = webase + g * B
        pltpu.make_async_copy(ew_hbm.at[pl.ds(e0, B)],
                              ew_v.at[pl.ds(bo * B, B)], sem).wait()
        pltpu.sync_copy(ew_v.at[pl.ds(bo * B, B)],
                        sew_acc.at[dst_v.at[bo]], add=True)
        pltpu.sync_copy(ones_v, deg_acc.at[dst_v.at[bo]], add=True)
        return carry
    lax.fori_loop(0, NCHUNK, _chunk, None)

    plsc.subcore_barrier()

    out0 = pl.multiple_of(c * N + r0, 8)
    pltpu.sync_copy(sew_acc.at[pl.ds(r0, STRIPE)],
                    sew_out.at[pl.ds(out0, STRIPE)])
    pltpu.sync_copy(deg_acc.at[pl.ds(r0, STRIPE)],
                    deg_out.at[pl.ds(out0, STRIPE)])

    @pl.when(s == NS - 1)
    def _copy_tail():
        t0 = pl.multiple_of(c * N + NS * STRIPE, 8)
        pltpu.sync_copy(sew_acc.at[pl.ds(NS * STRIPE, TAIL)],
                        sew_out.at[pl.ds(t0, TAIL)])
        pltpu.sync_copy(deg_acc.at[pl.ds(NS * STRIPE, TAIL)],
                        deg_out.at[pl.ds(t0, TAIL)])


def _dense_body(h1p_ref, sdp_ref, feat_ref,
                waug_ref, w1_ref, b1_ref, w2_ref, b2_ref,
                g1_ref, be1_ref, g2_ref, be2_ref, out_ref):
    f32 = jnp.float32
    h1 = h1p_ref[0] + h1p_ref[1]
    sd = sdp_ref[0] + sdp_ref[1]

    dn_t = (((1,), (1,)), ((), ()))
    dn_n = (((1,), (0,)), ((), ()))
    h2 = lax.dot_general(sd, waug_ref[...], dn_n,
                         preferred_element_type=f32)
    x = lax.dot_general(h1, w1_ref[...], dn_t,
                        preferred_element_type=f32)
    x = x + b1_ref[...] + h2

    m1 = jnp.mean(x, axis=0)
    v1 = jnp.mean((x - m1) ** 2, axis=0)
    x = (x - m1) * lax.rsqrt(v1 + 1e-5) * g1_ref[...] + be1_ref[...]
    x = jnp.maximum(x, 0.0)

    y = lax.dot_general(x, w2_ref[...], dn_t,
                        preferred_element_type=f32)
    y = y + b2_ref[...] + feat_ref[...]

    m2 = jnp.mean(y, axis=0)
    v2 = jnp.mean((y - m2) ** 2, axis=0)
    y = (y - m2) * lax.rsqrt(v2 + 1e-5) * g2_ref[...] + be2_ref[...]
    out_ref[...] = jnp.maximum(y, 0.0)


@jax.jit
def kernel(features, edge_w, W_bond, b_bond, W1, b1, W2, b2,
           bn1_gamma, bn1_beta, bn2_gamma, bn2_beta, edge_index):
    src2 = edge_index[0].astype(jnp.int32).reshape(E // B, B)
    dst2 = edge_index[1].astype(jnp.int32).reshape(E // B, B)
    waug = jnp.concatenate(
        [W_bond.T, b_bond[None, :], jnp.zeros((DE - 1, H), jnp.float32)],
        axis=0)

    h1_fn = pl.kernel(
        _h1_body,
        out_type=jax.ShapeDtypeStruct((NC * N, H), jnp.float32),
        mesh=_MESH,
        compiler_params=pltpu.CompilerParams(use_tc_tiling_on_sc=False),
        scratch_types=[
            pltpu.VMEM((2, B), jnp.int32),
            pltpu.VMEM((2, B), jnp.int32),
            pltpu.VMEM((2 * B, H), jnp.float32),
            pltpu.VMEM((16, H), jnp.float32),
            pltpu.VMEM_SHARED((N, H), jnp.float32),
            pltpu.SemaphoreType.DMA,
        ],
    )
    h1p = h1_fn(features, src2, dst2)

    sew_fn = pl.kernel(
        _sew_body,
        out_type=(
            jax.ShapeDtypeStruct((NC * N, DE), jnp.float32),
            jax.ShapeDtypeStruct((NC * N, DE), jnp.float32),
        ),
        mesh=_MESH,
        compiler_params=pltpu.CompilerParams(use_tc_tiling_on_sc=False),
        scratch_types=[
            pltpu.VMEM((2, B), jnp.int32),
            pltpu.VMEM((2 * B, DE), jnp.float32),
            pltpu.VMEM((B, DE), jnp.float32),
            pltpu.VMEM((16, DE), jnp.float32),
            pltpu.VMEM_SHARED((N, DE), jnp.float32),
            pltpu.VMEM_SHARED((N, DE), jnp.float32),
            pltpu.SemaphoreType.DMA,
        ],
    )
    sewp, degp = sew_fn(dst2, edge_w)

    sdp = jnp.concatenate(
        [sewp.reshape(NC, N, DE), degp.reshape(NC, N, DE)], axis=2)
    out = pl.pallas_call(
        _dense_body,
        out_shape=jax.ShapeDtypeStruct((N, H), jnp.float32),
    )(h1p.reshape(NC, N, H), sdp,
      features, waug, W1, b1, W2, b2,
      bn1_gamma, bn1_beta, bn2_gamma, bn2_beta)
    return out
